# Initial kernel scaffold; baseline (speedup 1.0000x reference)
import jax
import jax.numpy as jnp
from jax.experimental import pallas as pl


def kernel(x, operation):
    raise NotImplementedError("write your pallas kernel here")


# speedup vs baseline: 2584.4833x; 2584.4833x over previous
import jax
import jax.numpy as jnp
from jax import lax
from jax.experimental import pallas as pl
from jax.experimental.pallas import tpu as pltpu
from jax.experimental.pallas import tpu_sc as plsc

H = 128
WD = 128
ROWS = H + 2
COLS = 144
L = 16
NCH = WD // L


def _sc_body(xp_hbm, op_hbm, out_hbm, w_v, op_v, val_v, xpar_v):
    c = lax.axis_index("c")
    s = lax.axis_index("s")

    @pl.when(jnp.logical_and(c == 0, s == 0))
    def _():
        pltpu.sync_copy(xp_hbm, w_v)
        pltpu.sync_copy(op_hbm, op_v)
        zero_i = jnp.zeros((L,), jnp.int32)
        one_i = jnp.ones((L,), jnp.int32)
        two_i = jnp.full((L,), 2, jnp.int32)
        neg1_i = jnp.full((L,), -1, jnp.int32)
        last_i = jnp.full((L,), L - 1, jnp.int32)
        one_f = jnp.ones((L,), jnp.float32)
        two_f = jnp.full((L,), 2.0, jnp.float32)
        val_v[pl.ds(0, L)] = zero_i
        xpar_v[pl.ds(0, L)] = zero_i

        def row_body(row, carry_unused):
            rb = row * COLS
            maxc = neg1_i
            sumc = zero_i
            for q in range(NCH):
                o = q * L
                N = w_v[pl.ds(rb - COLS + o + 1, L)]
                NW = w_v[pl.ds(rb - COLS + o, L)]
                NE = w_v[pl.ds(rb - COLS + o + 2, L)]
                b = w_v[pl.ds(rb + o + 1, L)]
                E = w_v[pl.ds(rb + o + 2, L)]
                S = w_v[pl.ds(rb + COLS + o + 1, L)]
                SW = w_v[pl.ds(rb + COLS + o, L)]
                SE = w_v[pl.ds(rb + COLS + o + 2, L)]

                K = one_f - (N + E + S) + NE * N * E + SE * S * E
                M = NW * N + SW * S - one_f
                sgn = one_f - two_f * b
                a0 = jnp.where(sgn * K == one_f, one_i, zero_i)
                a1 = jnp.where(sgn * (K + M) == one_f, one_i, zero_i)
                bi = b.astype(jnp.int32)
                t0 = plsc.load_gather(op_v, [two_i * a0 + bi])
                t1 = plsc.load_gather(op_v, [two_i * a1 + bi])

                is_const = t0 == t1
                d = jnp.where(is_const, zero_i, t0)
                pos = lax.iota(jnp.int32, L) + jnp.full((L,), o, jnp.int32)
                kidx = jnp.maximum(plsc.cummax(jnp.where(is_const, pos,
                                                         neg1_i)), maxc)
                csum = plsc.cumsum(d) + sumc
                xpar = csum & one_i
                val_v[pl.ds(o + 1, L)] = t0
                xpar_v[pl.ds(o + 1, L)] = xpar
                gidx = kidx + one_i
                base = plsc.load_gather(val_v, [gidx])
                xk = plsc.load_gather(xpar_v, [gidx])
                cbits = (base + xpar + xk) & one_i
                w_v[pl.ds(rb + o + 1, L)] = cbits.astype(jnp.float32)
                maxc = kidx.at[last_i].get(mode="promise_in_bounds")
                sumc = csum.at[last_i].get(mode="promise_in_bounds")
            return carry_unused

        lax.fori_loop(1, H + 1, row_body, jnp.int32(0))
        pltpu.sync_copy(w_v, out_hbm)


_mesh = plsc.VectorSubcoreMesh(core_axis_name="c", subcore_axis_name="s")

_sc_call = pl.kernel(
    _sc_body,
    out_type=jax.ShapeDtypeStruct((ROWS * COLS,), jnp.float32),
    mesh=_mesh,
    scratch_types=[
        pltpu.VMEM((ROWS * COLS,), jnp.float32),
        pltpu.VMEM((L,), jnp.int32),
        pltpu.VMEM((COLS,), jnp.int32),
        pltpu.VMEM((COLS,), jnp.int32),
    ],
    compiler_params=pltpu.CompilerParams(needs_layout_passes=False),
)


@jax.jit
def kernel(x, operation):
    xp = jnp.zeros((ROWS, COLS), jnp.float32)
    xp = xp.at[1:H + 1, 1:WD + 1].set(x[0])
    opi = jnp.zeros((L,), jnp.int32).at[:4].set(operation.astype(jnp.int32))
    out = _sc_call(xp.reshape(ROWS * COLS), opi)
    return out.reshape(ROWS, COLS)[1:H + 1, 1:WD + 1][None]
